# Initial kernel scaffold; baseline (speedup 1.0000x reference)
import jax
import jax.numpy as jnp
from jax.experimental import pallas as pl


def kernel(x, W1, b1, W2, b2, idx1, idx2, out_idx1, out_idx2):
    raise NotImplementedError("write your pallas kernel here")


# speedup vs baseline: 9.7086x; 9.7086x over previous
import functools

import jax
import jax.numpy as jnp
from jax import lax
from jax.experimental import pallas as pl
from jax.experimental.pallas import tpu as pltpu
from jax.experimental.pallas import tpu_sc as plsc

_B, _IN, _O1, _O2, _F = 1024, 1024, 2048, 2048, 32
_NC, _NS, _L = 2, 16, 16
_NW = _NC * _NS
_R1 = _O1 // _NW
_R2 = _O2 // _NW
_CH = 32


def _densify_body(idx1_hbm, w1_hbm, idx2_hbm, w2_hbm, d1_hbm, d2_hbm,
                  idx1_v, w1_v, idx2_v, w2_v, acc1, acc2):
    wid = lax.axis_index("s") * _NC + lax.axis_index("c")
    lane = lax.iota(jnp.int32, _L)
    zero = jnp.zeros((_L,), jnp.float32)

    base1 = wid * _R1
    base2 = wid * _R2
    pltpu.sync_copy(idx1_hbm.at[pl.ds(base1 * _F, _R1 * _F)], idx1_v)
    pltpu.sync_copy(w1_hbm.at[pl.ds(base1 * _F, _R1 * _F)], w1_v)
    pltpu.sync_copy(idx2_hbm.at[pl.ds(base2 * _F, _R2 * _F)], idx2_v)
    pltpu.sync_copy(w2_hbm.at[pl.ds(base2 * _F, _R2 * _F)], w2_v)

    def layer(idx_v, w_v, acc, d_hbm, row_base, ncols, offset, nrows):
        for c in range(nrows // _CH):
            nvec = (_CH * ncols) // _L

            def zbody(i, carry):
                for k in range(8):
                    acc[pl.ds((i * 8 + k) * _L, _L)] = zero
                return carry

            lax.fori_loop(0, nvec // 8, zbody, 0)
            for g in range(_CH // _L):
                rows = lane + (c * _CH + g * _L)
                arow = lane + g * _L
                for f in range(_F):
                    src = rows * _F + f
                    col = plsc.load_gather(idx_v, [src]) - offset
                    wv = plsc.load_gather(w_v, [src])
                    addr = arow * ncols + col
                    plsc.addupdate_scatter(acc, [addr], wv)
            dst = (row_base + c * _CH) * ncols
            pltpu.sync_copy(acc, d_hbm.at[pl.ds(dst, _CH * ncols)])

    layer(idx1_v, w1_v, acc1, d1_hbm, base1, _IN, 1, _R1)
    layer(idx2_v, w2_v, acc2, d2_hbm, base2, _O1, 1 + _IN, _R2)


_densify = pl.kernel(
    _densify_body,
    out_type=[
        jax.ShapeDtypeStruct((_O1 * _IN,), jnp.float32),
        jax.ShapeDtypeStruct((_O2 * _O1,), jnp.float32),
    ],
    mesh=plsc.VectorSubcoreMesh(core_axis_name="c", subcore_axis_name="s"),
    compiler_params=pltpu.CompilerParams(needs_layout_passes=False),
    scratch_types=[
        pltpu.VMEM((_R1 * _F,), jnp.int32),
        pltpu.VMEM((_R1 * _F,), jnp.float32),
        pltpu.VMEM((_R2 * _F,), jnp.int32),
        pltpu.VMEM((_R2 * _F,), jnp.float32),
        pltpu.VMEM((_CH * _IN,), jnp.float32),
        pltpu.VMEM((_CH * _O1,), jnp.float32),
    ],
)


def _mm_body(x_ref, d1_ref, d2_ref, b1_ref, b2_ref, o_ref):
    h = lax.dot_general(x_ref[...], d1_ref[...], (((1,), (1,)), ((), ())),
                        preferred_element_type=jnp.float32)
    h = jnp.maximum(h + b1_ref[...], 0.0)
    y = lax.dot_general(h, d2_ref[...], (((1,), (1,)), ((), ())),
                        preferred_element_type=jnp.float32)
    o_ref[...] = y + b2_ref[...]


_BM = 256

_mm = pl.pallas_call(
    _mm_body,
    grid=(_B // _BM,),
    in_specs=[
        pl.BlockSpec((_BM, _IN), lambda i: (i, 0)),
        pl.BlockSpec((_O1, _IN), lambda i: (0, 0)),
        pl.BlockSpec((_O2, _O1), lambda i: (0, 0)),
        pl.BlockSpec((1, _O1), lambda i: (0, 0)),
        pl.BlockSpec((1, _O2), lambda i: (0, 0)),
    ],
    out_specs=pl.BlockSpec((_BM, _O2), lambda i: (i, 0)),
    out_shape=jax.ShapeDtypeStruct((_B, _O2), jnp.float32),
)


def kernel(x, W1, b1, W2, b2, idx1, idx2, out_idx1, out_idx2):
    d1_flat, d2_flat = _densify(idx1.reshape(-1), W1.reshape(-1),
                                idx2.reshape(-1), W2.reshape(-1))
    d1 = d1_flat.reshape(_O1, _IN)
    d2 = d2_flat.reshape(_O2, _O1)
    return _mm(x, d1, d2, b1.reshape(1, _O1), b2.reshape(1, _O2))


# speedup vs baseline: 13.7795x; 1.4193x over previous
import jax
import jax.numpy as jnp
from jax import lax
from jax.experimental import pallas as pl
from jax.experimental.pallas import tpu as pltpu
from jax.experimental.pallas import tpu_sc as plsc

_B, _IN, _O1, _O2, _F = 1024, 1024, 2048, 2048, 32
_NC, _NS, _L = 2, 16, 16
_NW = _NC * _NS
_R = _O1 // _NW
_CH = 16


def _densify_body(idx1_hbm, w1_hbm, idx2_hbm, w2_hbm, d1_hbm, d2_hbm,
                  idx1_v, w1_v, idx2_v, w2_v, acc1, acc2, sem_a, sem_b):
    wid = lax.axis_index("s") * _NC + lax.axis_index("c")
    lane = lax.iota(jnp.int32, _L)
    zero = jnp.zeros((_L,), jnp.float32)
    base = wid * _R

    pltpu.sync_copy(idx1_hbm.at[pl.ds(base * _F, _R * _F)], idx1_v)
    pltpu.sync_copy(w1_hbm.at[pl.ds(base * _F, _R * _F)], w1_v)
    pltpu.sync_copy(idx2_hbm.at[pl.ds(base * _F, _R * _F)], idx2_v)
    pltpu.sync_copy(w2_hbm.at[pl.ds(base * _F, _R * _F)], w2_v)

    sems = (sem_a, sem_b)

    def layer(idx_v, w_v, acc, d_hbm, ncols, offset):
        pend = [None, None]
        for c in range(_R // _CH):
            buf = c % 2
            if pend[buf] is not None:
                pend[buf].wait()

            def zbody(j, carry):
                for r in range(_CH):
                    for k in range(2):
                        acc[buf, r, pl.ds((j * 2 + k) * _L, _L)] = zero
                return carry

            lax.fori_loop(0, ncols // (2 * _L), zbody, 0)
            for f in range(_F):
                src = (lane + c * _CH) * _F + f
                col = plsc.load_gather(idx_v, [src]) - offset
                wv = plsc.load_gather(w_v, [src])
                plsc.addupdate_scatter(acc.at[buf], [lane, col], wv)
            pend[buf] = pltpu.async_copy(
                acc.at[buf], d_hbm.at[pl.ds(base + c * _CH, _CH)], sems[buf])
        for p in pend:
            if p is not None:
                p.wait()

    layer(idx1_v, w1_v, acc1, d1_hbm, _IN, 1)
    layer(idx2_v, w2_v, acc2, d2_hbm, _O1, 1 + _IN)


_densify = pl.kernel(
    _densify_body,
    out_type=[
        jax.ShapeDtypeStruct((_O1, _IN), jnp.float32),
        jax.ShapeDtypeStruct((_O2, _O1), jnp.float32),
    ],
    mesh=plsc.VectorSubcoreMesh(core_axis_name="c", subcore_axis_name="s"),
    compiler_params=pltpu.CompilerParams(needs_layout_passes=False),
    scratch_types=[
        pltpu.VMEM((_R * _F,), jnp.int32),
        pltpu.VMEM((_R * _F,), jnp.float32),
        pltpu.VMEM((_R * _F,), jnp.int32),
        pltpu.VMEM((_R * _F,), jnp.float32),
        pltpu.VMEM((2, _CH, _IN), jnp.float32),
        pltpu.VMEM((2, _CH, _O1), jnp.float32),
        pltpu.SemaphoreType.DMA,
        pltpu.SemaphoreType.DMA,
    ],
)


def _mm_body(x_ref, d1_ref, d2_ref, b1_ref, b2_ref, o_ref):
    h = lax.dot_general(x_ref[...], d1_ref[...], (((1,), (1,)), ((), ())),
                        preferred_element_type=jnp.float32)
    h = jnp.maximum(h + b1_ref[...], 0.0)
    y = lax.dot_general(h, d2_ref[...], (((1,), (1,)), ((), ())),
                        preferred_element_type=jnp.float32)
    o_ref[...] = y + b2_ref[...]


_BM = 256

_mm = pl.pallas_call(
    _mm_body,
    grid=(_B // _BM,),
    in_specs=[
        pl.BlockSpec((_BM, _IN), lambda i: (i, 0)),
        pl.BlockSpec((_O1, _IN), lambda i: (0, 0)),
        pl.BlockSpec((_O2, _O1), lambda i: (0, 0)),
        pl.BlockSpec((1, _O1), lambda i: (0, 0)),
        pl.BlockSpec((1, _O2), lambda i: (0, 0)),
    ],
    out_specs=pl.BlockSpec((_BM, _O2), lambda i: (i, 0)),
    out_shape=jax.ShapeDtypeStruct((_B, _O2), jnp.float32),
)


def kernel(x, W1, b1, W2, b2, idx1, idx2, out_idx1, out_idx2):
    d1, d2 = _densify(idx1.reshape(-1), W1.reshape(-1),
                      idx2.reshape(-1), W2.reshape(-1))
    return _mm(x, d1, d2, b1.reshape(1, _O1), b2.reshape(1, _O2))
